# Initial kernel scaffold; baseline (speedup 1.0000x reference)
import jax
import jax.numpy as jnp
from jax.experimental import pallas as pl


def kernel(x, Wr, br, W12, W3):
    raise NotImplementedError("write your pallas kernel here")


# speedup vs baseline: 2.6079x; 2.6079x over previous
import jax
import jax.numpy as jnp
from jax.experimental import pallas as pl
from jax.experimental.pallas import tpu as pltpu

DIM = 768
HID = 2048
NE = 8
S = 2048
CH = 512
HC = HID // CH


def _moe_body(x_ref, wr_ref, br_ref, w12a_ref, w12b_ref, w3_ref,
              out_ref, aux_ref, xb_ref, w8_ref):
    e = pl.program_id(0)
    j = pl.program_id(1)

    @pl.when((e == 0) & (j == 0))
    def _router():
        xf = x_ref[...]
        xb_ref[...] = xf.astype(jnp.bfloat16)
        logits = jax.lax.dot_general(
            xf, wr_ref[...], (((1,), (0,)), ((), ())),
            preferred_element_type=jnp.float32) + br_ref[...]
        aux_ref[...] = jnp.sum(logits * logits, axis=(0, 1),
                               keepdims=True) * (0.01 / (S * NE))
        scores = jax.nn.sigmoid(logits)
        iota = jax.lax.broadcasted_iota(jnp.int32, (S, NE), 1)
        v1 = jnp.max(scores, axis=1, keepdims=True)
        i1 = jnp.min(jnp.where(scores == v1, iota, NE), axis=1, keepdims=True)
        masked = jnp.where(iota == i1, -jnp.inf, scores)
        v2 = jnp.max(masked, axis=1, keepdims=True)
        i2 = jnp.min(jnp.where(masked == v2, iota, NE), axis=1, keepdims=True)
        sel = (iota == i1) | (iota == i2)
        w8_ref[...] = jnp.where(sel, scores, 0.0) / (v1 + v2 + 1e-6)
        out_ref[...] = jnp.zeros((S, DIM), jnp.float32)

    xb = xb_ref[...]
    u = jax.lax.dot_general(xb, w12a_ref[0].astype(jnp.bfloat16),
                            (((1,), (0,)), ((), ())),
                            preferred_element_type=jnp.float32)
    v = jax.lax.dot_general(xb, w12b_ref[0].astype(jnp.bfloat16),
                            (((1,), (0,)), ((), ())),
                            preferred_element_type=jnp.float32)
    g = (u * jax.nn.sigmoid(u) * v).astype(jnp.bfloat16)
    contrib = jax.lax.dot_general(g, w3_ref[0].astype(jnp.bfloat16),
                                  (((1,), (0,)), ((), ())),
                                  preferred_element_type=jnp.float32)
    iota = jax.lax.broadcasted_iota(jnp.int32, (S, NE), 1)
    wcol = jnp.sum(jnp.where(iota == e, w8_ref[...], 0.0),
                   axis=1, keepdims=True)
    out_ref[...] += contrib * wcol


def kernel(x, Wr, br, W12, W3):
    x2 = x.reshape(S, DIM)
    br2 = br.reshape(1, NE)
    out, aux = pl.pallas_call(
        _moe_body,
        grid=(NE, HC),
        in_specs=[
            pl.BlockSpec((S, DIM), lambda e, j: (0, 0)),
            pl.BlockSpec((DIM, NE), lambda e, j: (0, 0)),
            pl.BlockSpec((1, NE), lambda e, j: (0, 0)),
            pl.BlockSpec((1, DIM, CH), lambda e, j: (e, 0, j)),
            pl.BlockSpec((1, DIM, CH), lambda e, j: (e, 0, j + HC)),
            pl.BlockSpec((1, CH, DIM), lambda e, j: (e, j, 0)),
        ],
        out_specs=[
            pl.BlockSpec((S, DIM), lambda e, j: (0, 0)),
            pl.BlockSpec((1, 1), lambda e, j: (0, 0)),
        ],
        out_shape=[
            jax.ShapeDtypeStruct((S, DIM), jnp.float32),
            jax.ShapeDtypeStruct((1, 1), jnp.float32),
        ],
        scratch_shapes=[
            pltpu.VMEM((S, DIM), jnp.bfloat16),
            pltpu.VMEM((S, NE), jnp.float32),
        ],
        compiler_params=pltpu.CompilerParams(
            dimension_semantics=("arbitrary", "arbitrary")),
    )(x2, Wr, br2, W12, W12, W3)
    return out.reshape(1, S, DIM), aux.reshape(())


# speedup vs baseline: 3.0265x; 1.1605x over previous
import functools

import jax
import jax.numpy as jnp
from jax import lax
from jax.experimental import pallas as pl
from jax.experimental.pallas import tpu as pltpu
from jax.experimental.pallas import tpu_sc as plsc

DIM = 768
HID = 2048
NE = 8
S = 2048
NP = 2 * S
CH = 512
HC = HID // CH
C = 4608
RT = 256


def _router_body(x_ref, wr_ref, br_ref, eid_ref, wts_ref, aux_ref):
    xf = x_ref[...]
    logits = jax.lax.dot_general(
        xf, wr_ref[...], (((1,), (0,)), ((), ())),
        preferred_element_type=jnp.float32) + br_ref[...]
    aux_ref[...] = jnp.sum(logits * logits, axis=(0, 1),
                           keepdims=True) * (0.01 / (S * NE))
    scores = jax.nn.sigmoid(logits)
    iota = jax.lax.broadcasted_iota(jnp.int32, (S, NE), 1)
    v1 = jnp.max(scores, axis=1, keepdims=True)
    i1 = jnp.min(jnp.where(scores == v1, iota, NE), axis=1, keepdims=True)
    masked = jnp.where(iota == i1, -jnp.inf, scores)
    v2 = jnp.max(masked, axis=1, keepdims=True)
    i2 = jnp.min(jnp.where(masked == v2, iota, NE), axis=1, keepdims=True)
    eid_ref[...] = jnp.concatenate([i1, i2], axis=1)
    den = 1.0 / (v1 + v2 + 1e-6)
    wts_ref[...] = jnp.concatenate([v1 * den, v2 * den], axis=1)


def _router(x2, Wr, br2):
    return pl.pallas_call(
        _router_body,
        out_shape=[
            jax.ShapeDtypeStruct((S, 2), jnp.int32),
            jax.ShapeDtypeStruct((S, 2), jnp.float32),
            jax.ShapeDtypeStruct((1, 1), jnp.float32),
        ],
    )(x2, Wr, br2)


_NTILES = 16
_TPP = NP // _NTILES
_TTOK = S // _NTILES


def _group_body(eidf, pos0, pos1, offs, acnt,
                e_v, st_v, p01_v, hists_sp, allh_v):
    tid = lax.axis_index("s")
    lane = lax.iota(jnp.int32, 16)

    pltpu.sync_copy(eidf.at[pl.ds(tid * _TPP, _TPP)], e_v)

    def _hist_chunk(c, hist):
        ev = e_v[pl.ds(c * 16, 16)]
        for b in range(NE):
            cnt = jnp.sum((ev == b).astype(jnp.int32))
            hist = hist + jnp.where(lane == b, cnt, 0)
        return hist

    hist = lax.fori_loop(0, _TPP // 16, _hist_chunk,
                         jnp.zeros((16,), jnp.int32))
    st_v[...] = hist
    pltpu.sync_copy(st_v, hists_sp.at[pl.ds(tid * 16, 16)])
    plsc.subcore_barrier()
    pltpu.sync_copy(hists_sp, allh_v)

    def _acc(t, carry):
        tot, pref = carry
        row = allh_v[pl.ds(t * 16, 16)]
        tv = jnp.zeros((16,), jnp.int32) + t
        tot = tot + row
        pref = pref + jnp.where(tv < tid, row, 0)
        return tot, pref

    totals, prefix = lax.fori_loop(
        0, _NTILES, _acc,
        (jnp.zeros((16,), jnp.int32), jnp.zeros((16,), jnp.int32)))
    aligned = jnp.bitwise_and(totals + 7, -8)
    incl = plsc.cumsum(aligned)
    offs_v = incl - aligned
    bases0 = offs_v + prefix

    @pl.when(tid == 0)
    def _emit_meta():
        st_v[...] = offs_v
        pltpu.sync_copy(st_v, offs)
        st_v[...] = aligned
        pltpu.sync_copy(st_v, acnt)

    def _place_chunk(c, bases):
        ev = e_v[pl.ds(c * 16, 16)]
        pidx = lane + c * 16
        dest = (pidx & 1) * _TTOK + (pidx >> 1)
        pos = jnp.zeros((16,), jnp.int32)
        for b in range(NE):
            m = ev == b
            incl_b = plsc.cumsum(m.astype(jnp.int32))
            base_b = jnp.sum(jnp.where(lane == b, bases, 0))
            pos = jnp.where(m, incl_b + (base_b - 1), pos)
            cnt = incl_b[15]
            bases = bases + jnp.where(lane == b, cnt, 0)
        plsc.store_scatter(p01_v, [dest], pos)
        return bases

    lax.fori_loop(0, _TPP // 16, _place_chunk, bases0)
    t0 = tid * _TTOK
    pltpu.sync_copy(p01_v.at[pl.ds(0, _TTOK)], pos0.at[pl.ds(t0, _TTOK)])
    pltpu.sync_copy(p01_v.at[pl.ds(_TTOK, _TTOK)], pos1.at[pl.ds(t0, _TTOK)])


def _grouping(eidf):
    mesh = plsc.VectorSubcoreMesh(core_axis_name="c", subcore_axis_name="s",
                                  num_cores=1)
    return pl.kernel(
        _group_body,
        out_type=[
            jax.ShapeDtypeStruct((S,), jnp.int32),
            jax.ShapeDtypeStruct((S,), jnp.int32),
            jax.ShapeDtypeStruct((16,), jnp.int32),
            jax.ShapeDtypeStruct((16,), jnp.int32),
        ],
        mesh=mesh,
        scratch_types=[
            pltpu.VMEM((_TPP,), jnp.int32),
            pltpu.VMEM((16,), jnp.int32),
            pltpu.VMEM((_TPP,), jnp.int32),
            pltpu.VMEM_SHARED((_NTILES * 16,), jnp.int32),
            pltpu.VMEM((_NTILES * 16,), jnp.int32),
        ],
        compiler_params=pltpu.CompilerParams(needs_layout_passes=False),
    )(eidf)


_NW = 32
_DTOK = S // _NW


def _dispatch_body(x_hbm, pos0, pos1, xg, xr_v, p0_v, p1_v, sem):
    wid = lax.axis_index("c") * 16 + lax.axis_index("s")
    t0 = wid * _DTOK
    pltpu.sync_copy(pos0.at[pl.ds(t0, _DTOK)], p0_v)
    pltpu.sync_copy(pos1.at[pl.ds(t0, _DTOK)], p1_v)
    pltpu.sync_copy(x_hbm.at[pl.ds(t0, _DTOK)], xr_v)
    c0 = pltpu.async_copy(xr_v, xg.at[p0_v], sem)
    c1 = pltpu.async_copy(xr_v, xg.at[p1_v], sem)
    c0.wait()
    c1.wait()


def _dispatch(x2, pos0, pos1):
    mesh = plsc.VectorSubcoreMesh(core_axis_name="c", subcore_axis_name="s")
    return pl.kernel(
        _dispatch_body,
        out_type=jax.ShapeDtypeStruct((C, DIM), jnp.float32),
        mesh=mesh,
        scratch_types=[
            pltpu.VMEM((_DTOK, DIM), jnp.float32),
            pltpu.VMEM((_DTOK,), jnp.int32),
            pltpu.VMEM((_DTOK,), jnp.int32),
            pltpu.SemaphoreType.DMA,
        ],
        compiler_params=pltpu.CompilerParams(needs_layout_passes=False),
    )(x2, pos0, pos1)


def _expert_body(offs_s, acnt_s, xg_ref, w12a_ref, w12b_ref, w3_ref, yg_ref):
    e = pl.program_id(0)
    j = pl.program_id(1)
    off = offs_s[e]
    nt = (acnt_s[e] + (RT - 1)) // RT
    w12a = w12a_ref[0].astype(jnp.bfloat16)
    w12b = w12b_ref[0].astype(jnp.bfloat16)
    w3 = w3_ref[0].astype(jnp.bfloat16)

    def _tile(t, _):
        r0 = pl.multiple_of(off + t * RT, 8)
        xt = xg_ref[pl.ds(r0, RT), :].astype(jnp.bfloat16)
        u = jax.lax.dot_general(xt, w12a, (((1,), (0,)), ((), ())),
                                preferred_element_type=jnp.float32)
        v = jax.lax.dot_general(xt, w12b, (((1,), (0,)), ((), ())),
                                preferred_element_type=jnp.float32)
        g = (u * jax.nn.sigmoid(u) * v).astype(jnp.bfloat16)
        contrib = jax.lax.dot_general(g, w3, (((1,), (0,)), ((), ())),
                                      preferred_element_type=jnp.float32)

        @pl.when(j == 0)
        def _init():
            yg_ref[pl.ds(r0, RT), :] = contrib

        @pl.when(j != 0)
        def _accum():
            yg_ref[pl.ds(r0, RT), :] += contrib

        return _

    lax.fori_loop(0, nt, _tile, 0)


def _experts(offs, acnt, xg, W12, W3):
    grid_spec = pltpu.PrefetchScalarGridSpec(
        num_scalar_prefetch=2,
        grid=(NE, HC),
        in_specs=[
            pl.BlockSpec((C, DIM), lambda e, j, *_: (0, 0)),
            pl.BlockSpec((1, DIM, CH), lambda e, j, *_: (e, 0, j)),
            pl.BlockSpec((1, DIM, CH), lambda e, j, *_: (e, 0, j + HC)),
            pl.BlockSpec((1, CH, DIM), lambda e, j, *_: (e, j, 0)),
        ],
        out_specs=pl.BlockSpec((C, DIM), lambda e, j, *_: (0, 0)),
    )
    return pl.pallas_call(
        _expert_body,
        grid_spec=grid_spec,
        out_shape=jax.ShapeDtypeStruct((C, DIM), jnp.float32),
        compiler_params=pltpu.CompilerParams(
            dimension_semantics=("arbitrary", "arbitrary")),
    )(offs, acnt, xg, W12, W12, W3)


def _combine_body(yg, pos0, pos1, wtsf, out, r0_v, r1_v, p0_v, p1_v, w_v,
                  sem):
    wid = lax.axis_index("c") * 16 + lax.axis_index("s")
    t0 = wid * _DTOK
    pltpu.sync_copy(pos0.at[pl.ds(t0, _DTOK)], p0_v)
    pltpu.sync_copy(pos1.at[pl.ds(t0, _DTOK)], p1_v)
    pltpu.sync_copy(wtsf.at[pl.ds(2 * t0, 2 * _DTOK)], w_v)
    c0 = pltpu.async_copy(yg.at[p0_v], r0_v, sem)
    c1 = pltpu.async_copy(yg.at[p1_v], r1_v, sem)
    c0.wait()
    c1.wait()

    def _tok(t, _):
        i0 = jnp.zeros((16,), jnp.int32) + 2 * t
        w0 = plsc.load_gather(w_v, [i0])
        w1 = plsc.load_gather(w_v, [i0 + 1])
        for c in range(DIM // 16):
            sl = pl.ds(c * 16, 16)
            r0_v[t, sl] = w0 * r0_v[t, sl] + w1 * r1_v[t, sl]
        return _

    lax.fori_loop(0, _DTOK, _tok, 0)
    pltpu.sync_copy(r0_v, out.at[pl.ds(t0, _DTOK)])


def _combine(yg, pos0, pos1, wtsf):
    mesh = plsc.VectorSubcoreMesh(core_axis_name="c", subcore_axis_name="s")
    return pl.kernel(
        _combine_body,
        out_type=jax.ShapeDtypeStruct((S, DIM), jnp.float32),
        mesh=mesh,
        scratch_types=[
            pltpu.VMEM((_DTOK, DIM), jnp.float32),
            pltpu.VMEM((_DTOK, DIM), jnp.float32),
            pltpu.VMEM((_DTOK,), jnp.int32),
            pltpu.VMEM((_DTOK,), jnp.int32),
            pltpu.VMEM((2 * _DTOK,), jnp.float32),
            pltpu.SemaphoreType.DMA,
        ],
        compiler_params=pltpu.CompilerParams(needs_layout_passes=False),
    )(yg, pos0, pos1, wtsf)


def kernel(x, Wr, br, W12, W3):
    x2 = x.reshape(S, DIM)
    br2 = br.reshape(1, NE)
    eid, wts, aux = _router(x2, Wr, br2)
    pos0, pos1, offs, acnt = _grouping(eid.reshape(NP))
    xg = _dispatch(x2, pos0, pos1)
    yg = _experts(offs, acnt, xg, W12, W3)
    out = _combine(yg, pos0, pos1, wts.reshape(NP))
    return out.reshape(1, S, DIM), aux.reshape(())
